# baseline (speedup 1.0000x reference)
import functools

import jax
import jax.numpy as jnp
from jax import lax
from jax.experimental import pallas as pl
from jax.experimental.pallas import tpu as pltpu
from jax.experimental.pallas import tpu_sc as plsc


@functools.cache
def _make_sc_gather(V, D, S, B):
    info = plsc.get_sparse_core_info()
    NC, NS = info.num_cores, info.num_subcores
    NW = NC * NS
    assert B % NW == 0 and S % 2 == 0
    BW = B // NW
    mesh = plsc.VectorSubcoreMesh(core_axis_name="c", subcore_axis_name="s")

    @functools.partial(
        pl.kernel,
        mesh=mesh,
        compiler_params=pltpu.CompilerParams(use_tc_tiling_on_sc=False),
        out_type=jax.ShapeDtypeStruct((B, S * D), jnp.float32),
        scratch_types=[
            pltpu.VMEM((S, BW), jnp.int32),
            pltpu.VMEM((BW, D), jnp.float32),
            pltpu.VMEM((BW, D), jnp.float32),
            pltpu.SemaphoreType.DMA,
            pltpu.SemaphoreType.DMA,
            pltpu.SemaphoreType.DMA,
            pltpu.SemaphoreType.DMA,
        ],
    )
    def sc_gather(table_hbm, idx_hbm, out_hbm, idx_v, rows0, rows1,
                  gs0, gs1, ws0, ws1):
        wid = lax.axis_index("s") * NC + lax.axis_index("c")
        b0 = wid * BW
        pltpu.sync_copy(idx_hbm.at[:, pl.ds(b0, BW)], idx_v)

        rows, gs, ws = [rows0, rows1], [gs0, gs1], [ws0, ws1]

        def wb_slice(j):
            return out_hbm.at[pl.ds(b0, BW), pl.ds(j * D, D)]

        def body(jj, carry):
            j0 = jj * 2
            for k in range(2):
                j = j0 + k

                @pl.when(jj >= 1)
                def _():
                    pltpu.make_async_copy(rows[k], wb_slice(j), ws[k]).wait()

                pltpu.async_copy(
                    table_hbm.at[idx_v.at[j]], rows[k], gs[k]).wait()
                pltpu.make_async_copy(rows[k], wb_slice(j), ws[k]).start()
            return carry

        lax.fori_loop(0, S // 2, body, 0)
        for k in range(2):
            pltpu.make_async_copy(rows[k], wb_slice(S - 2 + k), ws[k]).wait()

    return sc_gather


def _mlp_body(h_ref, w1_ref, b1_ref, w2_ref, b2_ref, o_ref):
    h = h_ref[...].astype(jnp.bfloat16)
    z = jnp.dot(h, w1_ref[...], preferred_element_type=jnp.float32)
    z = jnp.maximum(z + b1_ref[...], 0.0)
    logits = jnp.dot(z, w2_ref[...], preferred_element_type=jnp.float32)
    logits = logits + b2_ref[...]
    m = jnp.max(logits, axis=-1, keepdims=True)
    e = jnp.exp(logits - m)
    o_ref[...] = e / jnp.sum(e, axis=-1, keepdims=True)


@functools.cache
def _make_tc_mlp(B, K, N1, N2, BM):
    grid = (B // BM,)
    return pl.pallas_call(
        _mlp_body,
        grid=grid,
        in_specs=[
            pl.BlockSpec((BM, K), lambda i: (i, 0)),
            pl.BlockSpec((K, N1), lambda i: (0, 0)),
            pl.BlockSpec((1, N1), lambda i: (0, 0)),
            pl.BlockSpec((N1, N2), lambda i: (0, 0)),
            pl.BlockSpec((1, N2), lambda i: (0, 0)),
        ],
        out_specs=pl.BlockSpec((BM, N2), lambda i: (i, 0)),
        out_shape=jax.ShapeDtypeStruct((B, N2), jnp.float32),
    )


def kernel(x, emb, W1, b1, W2, b2):
    Bx, S = x.shape
    V, D = emb.shape
    K = S * D
    N1 = W1.shape[1]
    N2 = W2.shape[1]

    idx_t = x.T.astype(jnp.int32)
    h = _make_sc_gather(V, D, S, Bx)(emb, idx_t)
    out = _make_tc_mlp(Bx, K, N1, N2, 1024)(
        h, W1.astype(jnp.bfloat16), b1.reshape(1, N1), W2,
        b2.reshape(1, N2))
    return out


# speedup vs baseline: 1.1357x; 1.1357x over previous
import functools

import jax
import jax.numpy as jnp
from jax import lax
from jax.experimental import pallas as pl
from jax.experimental.pallas import tpu as pltpu
from jax.experimental.pallas import tpu_sc as plsc


@functools.cache
def _make_sc_gather(V, D, B):
    info = plsc.get_sparse_core_info()
    NC, NS = info.num_cores, info.num_subcores
    NW = NC * NS
    assert B % NW == 0
    b_per_w = B // NW
    CH = min(1280, b_per_w)
    assert b_per_w % CH == 0 and CH % 8 == 0
    n_chunks = b_per_w // CH
    mesh = plsc.VectorSubcoreMesh(core_axis_name="c", subcore_axis_name="s")

    @functools.partial(
        pl.kernel,
        mesh=mesh,
        compiler_params=pltpu.CompilerParams(use_tc_tiling_on_sc=False),
        out_type=jax.ShapeDtypeStruct((B, D), jnp.float32),
        scratch_types=[
            pltpu.VMEM((b_per_w,), jnp.int32),
            pltpu.VMEM((CH, D), jnp.float32),
            pltpu.VMEM((CH, D), jnp.float32),
            pltpu.SemaphoreType.DMA,
            pltpu.SemaphoreType.DMA,
            pltpu.SemaphoreType.DMA,
            pltpu.SemaphoreType.DMA,
        ],
    )
    def sc_gather(table_hbm, idx_hbm, out_hbm, idx_v, rows0, rows1,
                  gs0, gs1, ws0, ws1):
        wid = lax.axis_index("s") * NC + lax.axis_index("c")
        base = wid * b_per_w
        pltpu.sync_copy(idx_hbm.at[pl.ds(base, b_per_w)], idx_v)

        rows, gs, ws = [rows0, rows1], [gs0, gs1], [ws0, ws1]
        gcop, wcop = [None, None], [None, None]

        def start_gather(c):
            gcop[c % 2] = pltpu.async_copy(
                table_hbm.at[idx_v.at[pl.ds(c * CH, CH)]],
                rows[c % 2], gs[c % 2])

        start_gather(0)
        for c in range(n_chunks):
            if c >= 1:
                wcop[(c - 1) % 2].wait()
            if c + 1 < n_chunks:
                start_gather(c + 1)
            gcop[c % 2].wait()
            wcop[c % 2] = pltpu.async_copy(
                rows[c % 2], out_hbm.at[pl.ds(base + c * CH, CH)], ws[c % 2])
        wcop[(n_chunks - 1) % 2].wait()

    return sc_gather


def _mlp_body(h_ref, w1_ref, b1_ref, w2_ref, b2_ref, o_ref):
    h = h_ref[...].astype(jnp.bfloat16)
    z = jnp.dot(h, w1_ref[...], preferred_element_type=jnp.float32)
    z = jnp.maximum(z + b1_ref[...], 0.0)
    logits = jnp.dot(z, w2_ref[...], preferred_element_type=jnp.float32)
    logits = logits + b2_ref[...]
    m = jnp.max(logits, axis=-1, keepdims=True)
    e = jnp.exp(logits - m)
    o_ref[...] = e / jnp.sum(e, axis=-1, keepdims=True)


@functools.cache
def _make_tc_mlp(B, K, N1, N2, BM):
    grid = (B // BM,)
    return pl.pallas_call(
        _mlp_body,
        grid=grid,
        in_specs=[
            pl.BlockSpec((BM, K), lambda i: (i, 0)),
            pl.BlockSpec((K, N1), lambda i: (0, 0)),
            pl.BlockSpec((1, N1), lambda i: (0, 0)),
            pl.BlockSpec((N1, N2), lambda i: (0, 0)),
            pl.BlockSpec((1, N2), lambda i: (0, 0)),
        ],
        out_specs=pl.BlockSpec((BM, N2), lambda i: (i, 0)),
        out_shape=jax.ShapeDtypeStruct((B, N2), jnp.float32),
    )


def kernel(x, emb, W1, b1, W2, b2):
    Bx, S = x.shape
    V, D = emb.shape
    K = S * D
    N1 = W1.shape[1]
    N2 = W2.shape[1]

    P = 4
    BP = Bx // P
    idx = x.reshape(-1).astype(jnp.int32)
    w1b = W1.astype(jnp.bfloat16)
    b1r, b2r = b1.reshape(1, N1), b2.reshape(1, N2)

    outs = []
    for p in range(P):
        h_flat = _make_sc_gather(V, D, BP * S)(
            emb, lax.dynamic_slice_in_dim(idx, p * BP * S, BP * S))
        h = h_flat.reshape(BP, K)
        outs.append(_make_tc_mlp(BP, K, N1, N2, min(1024, BP))(
            h, w1b, b1r, W2, b2r))
    return jnp.concatenate(outs, axis=0)


# speedup vs baseline: 1.3599x; 1.1974x over previous
import functools

import jax
import jax.numpy as jnp
from jax import lax
from jax.experimental import pallas as pl
from jax.experimental.pallas import tpu as pltpu
from jax.experimental.pallas import tpu_sc as plsc


@functools.cache
def _make_sc_gather(V, D, S, B, NT):
    info = plsc.get_sparse_core_info()
    NC, NS = info.num_cores, info.num_subcores
    NW = NC * NS
    NB = NW // 2
    assert B % NB == 0 and S % 2 == 0
    BW = B // NB
    SH = S // 2
    DPT = 128 // D
    mesh = plsc.VectorSubcoreMesh(core_axis_name="c", subcore_axis_name="s")

    @functools.partial(
        pl.kernel,
        mesh=mesh,
        compiler_params=pltpu.CompilerParams(use_tc_tiling_on_sc=False),
        out_type=jax.ShapeDtypeStruct((NT, B, 128), jnp.float32),
        scratch_types=[
            pltpu.VMEM((SH, BW), jnp.int32),
            pltpu.VMEM((BW, D), jnp.float32),
            pltpu.VMEM((BW, D), jnp.float32),
            pltpu.SemaphoreType.DMA,
            pltpu.SemaphoreType.DMA,
            pltpu.SemaphoreType.DMA,
            pltpu.SemaphoreType.DMA,
        ],
    )
    def sc_gather(table_hbm, idx_hbm, out_hbm, idx_v, rows0, rows1,
                  gs0, gs1, ws0, ws1):
        wid = lax.axis_index("s") * NC + lax.axis_index("c")
        bslice = wid % NB
        jhalf = wid // NB
        b0 = bslice * BW
        j_base = jhalf * SH
        pltpu.sync_copy(idx_hbm.at[pl.ds(j_base, SH), pl.ds(b0, BW)], idx_v)

        rows, gs, ws = [rows0, rows1], [gs0, gs1], [ws0, ws1]

        def wb_slice(j):
            jg = j_base + j
            return out_hbm.at[jg // DPT, pl.ds(b0, BW),
                              pl.ds((jg % DPT) * D, D)]

        def start_gather(j, k):
            return pltpu.make_async_copy(
                table_hbm.at[idx_v.at[j]], rows[k], gs[k])

        def body(jj, carry):
            j0 = jj * 2
            for k in range(2):
                @pl.when(jj >= 1)
                def _():
                    pltpu.make_async_copy(
                        rows[k], wb_slice(j0 + k), ws[k]).wait()
                start_gather(j0 + k, k).start()
            for k in range(2):
                pltpu.make_async_copy(
                    table_hbm.at[idx_v.at[j0 + k]], rows[k], gs[k]).wait()
                pltpu.make_async_copy(rows[k], wb_slice(j0 + k), ws[k]).start()
            return carry

        lax.fori_loop(0, SH // 2, body, 0)
        for k in range(2):
            pltpu.make_async_copy(rows[k], wb_slice(SH - 2 + k), ws[k]).wait()

    return sc_gather


@functools.cache
def _make_tc_mlp(B, NT, K, N1, N2, BM):
    def body(h_ref, w1_ref, b1_ref, w2_ref, b2_ref, o_ref):
        tail = K - (NT - 1) * 128
        acc = jnp.dot(h_ref[0].astype(jnp.bfloat16), w1_ref[0],
                      preferred_element_type=jnp.float32)
        for t in range(1, NT - 1):
            acc += jnp.dot(h_ref[t].astype(jnp.bfloat16), w1_ref[t],
                           preferred_element_type=jnp.float32)
        acc += jnp.dot(h_ref[NT - 1][:, :tail].astype(jnp.bfloat16),
                       w1_ref[NT - 1][:tail],
                       preferred_element_type=jnp.float32)
        z = jnp.maximum(acc + b1_ref[...], 0.0)
        logits = jnp.dot(z, w2_ref[...],
                         preferred_element_type=jnp.float32) + b2_ref[...]
        m = jnp.max(logits, axis=-1, keepdims=True)
        e = jnp.exp(logits - m)
        o_ref[...] = e / jnp.sum(e, axis=-1, keepdims=True)

    return pl.pallas_call(
        body,
        grid=(B // BM,),
        in_specs=[
            pl.BlockSpec((NT, BM, 128), lambda i: (0, i, 0)),
            pl.BlockSpec((NT, 128, N1), lambda i: (0, 0, 0)),
            pl.BlockSpec((1, N1), lambda i: (0, 0)),
            pl.BlockSpec((N1, N2), lambda i: (0, 0)),
            pl.BlockSpec((1, N2), lambda i: (0, 0)),
        ],
        out_specs=pl.BlockSpec((BM, N2), lambda i: (i, 0)),
        out_shape=jax.ShapeDtypeStruct((B, N2), jnp.float32),
    )


def kernel(x, emb, W1, b1, W2, b2):
    Bx, S = x.shape
    V, D = emb.shape
    K = S * D
    N1 = W1.shape[1]
    N2 = W2.shape[1]
    NT = (K + 127) // 128

    idx_t = x.T.astype(jnp.int32)
    w1p = jnp.zeros((NT * 128, N1), jnp.float32).at[:K].set(W1)
    w1p = w1p.reshape(NT, 128, N1).astype(jnp.bfloat16)
    b1r, b2r = b1.reshape(1, N1), b2.reshape(1, N2)

    P = 2
    BP = Bx // P
    outs = []
    for p in range(P):
        h3 = _make_sc_gather(V, D, S, BP, NT)(
            emb, lax.dynamic_slice_in_dim(idx_t, p * BP, BP, axis=1))
        outs.append(_make_tc_mlp(BP, NT, K, N1, N2, 1024)(
            h3, w1p, b1r, W2, b2r))
    return jnp.concatenate(outs, axis=0)
